# baseline (speedup 1.0000x reference)
import jax
import jax.numpy as jnp
from jax import lax
from jax.experimental import pallas as pl
from jax.experimental.pallas import tpu as pltpu

_N = 50000
_C = 256
_B = 512
_STEPS = 4
_G = 256
_RB = 2048
_NEG = -1e30
_NPAD = ((_N + _RB - 1) // _RB) * _RB


def _set2set_body(offs_ref, x_ref, lo_ref, hi_ref, wih_ref, whh_ref,
                  bih_ref, bhh_ref, out_ref, h_ref, c_ref, qst_ref, r_ref):
    h_ref[...] = jnp.zeros((_B, _C), jnp.float32)
    c_ref[...] = jnp.zeros((_B, _C), jnp.float32)
    qst_ref[...] = jnp.zeros((_B, 2 * _C), jnp.float32)
    b = bih_ref[...] + bhh_ref[...]
    eye_g = jnp.eye(_G, dtype=jnp.float32)
    ones_rb = jnp.ones((_RB, _G), jnp.bfloat16)

    for _ in range(_STEPS):
        gates = (
            jnp.dot(qst_ref[...], wih_ref[...], preferred_element_type=jnp.float32)
            + jnp.dot(h_ref[...], whh_ref[...], preferred_element_type=jnp.float32)
            + b
        )
        gi = jax.nn.sigmoid(gates[:, 0 * _C:1 * _C])
        gf = jax.nn.sigmoid(gates[:, 1 * _C:2 * _C])
        gg = jnp.tanh(gates[:, 2 * _C:3 * _C])
        go = jax.nn.sigmoid(gates[:, 3 * _C:4 * _C])
        c_ref[...] = gf * c_ref[...] + gi * gg
        h_ref[...] = go * jnp.tanh(c_ref[...])

        def grp_body(g, _):
            s0 = pl.multiple_of(g * _G, _G)
            start = offs_ref[s0]
            end = offs_ref[s0 + _G]
            base = (start // _RB) * _RB
            nb = (end - base + _RB - 1) // _RB
            qg = h_ref[pl.ds(s0, _G), :]
            lo = lo_ref[pl.ds(g, 1), :]
            hi = hi_ref[pl.ds(g, 1), :]

            qg16 = qg.astype(jnp.bfloat16)

            def one_block(r0, carry, valid):
                m, dcol, racc = carry
                xb = x_ref[pl.ds(r0, _RB), :]
                xb16 = xb.astype(jnp.bfloat16)
                e = lax.dot_general(xb16, qg16, (((1,), (1,)), ((), ())),
                                    preferred_element_type=jnp.float32)
                rows = r0 + lax.broadcasted_iota(jnp.int32, (_RB, 1), 0)
                oh = (rows >= lo) & (rows < hi) & valid
                em = jnp.where(oh, e, _NEG)
                m_new = jnp.maximum(m, jnp.max(em, axis=0, keepdims=True))
                scale = jnp.exp(jnp.maximum(m - m_new, -80.0))
                scol = lax.dot_general(eye_g, scale, (((1,), (1,)), ((), ())),
                                       preferred_element_type=jnp.float32)
                ex = jnp.where(oh, jnp.exp(jnp.clip(e - m_new, -80.0, 80.0)),
                               0.0)
                ex16 = ex.astype(jnp.bfloat16)
                dcol = dcol * scol + lax.dot_general(
                    ex16, ones_rb, (((0,), (0,)), ((), ())),
                    preferred_element_type=jnp.float32)[:, :1]
                racc = racc * scol + lax.dot_general(
                    ex16, xb16, (((0,), (0,)), ((), ())),
                    preferred_element_type=jnp.float32)
                return m_new, dcol, racc

            def blk(ib, carry):
                r0 = pl.multiple_of(base + ib * _RB, _RB)
                return one_block(r0, carry, True)

            _, dcol, racc = lax.fori_loop(
                0, nb, blk,
                (jnp.full((1, _G), _NEG, jnp.float32),
                 jnp.zeros((_G, 1), jnp.float32),
                 jnp.zeros((_G, _C), jnp.float32)))
            r_ref[pl.ds(s0, _G), :] = racc / (dcol + 1e-16)
            return 0

        lax.fori_loop(0, _B // _G, grp_body, 0)
        qst_ref[:, :_C] = h_ref[...]
        qst_ref[:, _C:] = r_ref[...]

    out_ref[...] = qst_ref[...]


@jax.jit
def kernel(x, batch, W_ih, W_hh, b_ih, b_hh):
    n_pad = ((_N + _RB - 1) // _RB) * _RB
    x_pad = jnp.pad(x, ((0, n_pad - _N), (0, 0)))
    offs = jnp.searchsorted(batch, jnp.arange(_B + 1, dtype=jnp.int32),
                            side="left").astype(jnp.int32)
    lo_g = offs[:_B].reshape(_B // _G, _G)
    hi_g = offs[1:_B + 1].reshape(_B // _G, _G)
    wih_t = W_ih.T
    whh_t = W_hh.T

    return pl.pallas_call(
        _set2set_body,
        out_shape=jax.ShapeDtypeStruct((_B, 2 * _C), jnp.float32),
        in_specs=[
            pl.BlockSpec(memory_space=pltpu.SMEM),
            pl.BlockSpec(memory_space=pltpu.VMEM),
            pl.BlockSpec(memory_space=pltpu.VMEM),
            pl.BlockSpec(memory_space=pltpu.VMEM),
            pl.BlockSpec(memory_space=pltpu.VMEM),
            pl.BlockSpec(memory_space=pltpu.VMEM),
            pl.BlockSpec(memory_space=pltpu.VMEM),
            pl.BlockSpec(memory_space=pltpu.VMEM),
        ],
        out_specs=pl.BlockSpec(memory_space=pltpu.VMEM),
        scratch_shapes=[
            pltpu.VMEM((_B, _C), jnp.float32),
            pltpu.VMEM((_B, _C), jnp.float32),
            pltpu.VMEM((_B, 2 * _C), jnp.float32),
            pltpu.VMEM((_B, _C), jnp.float32),
        ],
        compiler_params=pltpu.CompilerParams(
            vmem_limit_bytes=120 * 1024 * 1024,
        ),
    )(offs, x_pad, lo_g, hi_g, wih_t, whh_t, b_ih[None, :], b_hh[None, :])


# speedup vs baseline: 1.1379x; 1.1379x over previous
import jax
import jax.numpy as jnp
from jax import lax
from jax.experimental import pallas as pl
from jax.experimental.pallas import tpu as pltpu

_N = 50000
_C = 256
_B = 512
_STEPS = 4
_G = 128
_RB = 2048
_NEG = -1e30
_NPAD = ((_N + _RB - 1) // _RB) * _RB


def _set2set_body(offs_ref, x_ref, lo_ref, hi_ref, wih_ref, whh_ref,
                  bih_ref, bhh_ref, out_ref, h_ref, c_ref, qst_ref, r_ref):
    h_ref[...] = jnp.zeros((_B, _C), jnp.float32)
    c_ref[...] = jnp.zeros((_B, _C), jnp.float32)
    qst_ref[...] = jnp.zeros((_B, 2 * _C), jnp.float32)
    b = bih_ref[...] + bhh_ref[...]
    eye_g = jnp.eye(_G, dtype=jnp.float32)
    ones_rb = jnp.ones((_RB, _G), jnp.bfloat16)

    for _ in range(_STEPS):
        gates = (
            jnp.dot(qst_ref[...], wih_ref[...], preferred_element_type=jnp.float32)
            + jnp.dot(h_ref[...], whh_ref[...], preferred_element_type=jnp.float32)
            + b
        )
        gi = jax.nn.sigmoid(gates[:, 0 * _C:1 * _C])
        gf = jax.nn.sigmoid(gates[:, 1 * _C:2 * _C])
        gg = jnp.tanh(gates[:, 2 * _C:3 * _C])
        go = jax.nn.sigmoid(gates[:, 3 * _C:4 * _C])
        c_ref[...] = gf * c_ref[...] + gi * gg
        h_ref[...] = go * jnp.tanh(c_ref[...])

        def grp_body(g, _):
            s0 = pl.multiple_of(g * _G, _G)
            start = offs_ref[s0]
            end = offs_ref[s0 + _G]
            base = (start // _RB) * _RB
            nb = (end - base + _RB - 1) // _RB
            qg = h_ref[pl.ds(s0, _G), :]
            lo = lo_ref[pl.ds(g, 1), :]
            hi = hi_ref[pl.ds(g, 1), :]

            qg16 = qg.astype(jnp.bfloat16)

            def one_block(r0, carry, valid):
                m, dcol, racc = carry
                xb = x_ref[pl.ds(r0, _RB), :]
                xb16 = xb.astype(jnp.bfloat16)
                e = lax.dot_general(xb16, qg16, (((1,), (1,)), ((), ())),
                                    preferred_element_type=jnp.float32)
                rows = r0 + lax.broadcasted_iota(jnp.int32, (_RB, 1), 0)
                oh = (rows >= lo) & (rows < hi) & valid
                em = jnp.where(oh, e, _NEG)
                m_new = jnp.maximum(m, jnp.max(em, axis=0, keepdims=True))
                scale = jnp.exp(jnp.maximum(m - m_new, -80.0))
                scol = lax.dot_general(eye_g, scale, (((1,), (1,)), ((), ())),
                                       preferred_element_type=jnp.float32)
                ex = jnp.where(oh, jnp.exp(jnp.clip(e - m_new, -80.0, 80.0)),
                               0.0)
                ex16 = ex.astype(jnp.bfloat16)
                dcol = dcol * scol + lax.dot_general(
                    ex16, ones_rb, (((0,), (0,)), ((), ())),
                    preferred_element_type=jnp.float32)[:, :1]
                racc = racc * scol + lax.dot_general(
                    ex16, xb16, (((0,), (0,)), ((), ())),
                    preferred_element_type=jnp.float32)
                return m_new, dcol, racc

            def blk(ib, carry):
                r0 = pl.multiple_of(base + ib * _RB, _RB)
                return one_block(r0, carry, True)

            _, dcol, racc = lax.fori_loop(
                0, nb, blk,
                (jnp.full((1, _G), _NEG, jnp.float32),
                 jnp.zeros((_G, 1), jnp.float32),
                 jnp.zeros((_G, _C), jnp.float32)))
            r_ref[pl.ds(s0, _G), :] = racc / (dcol + 1e-16)
            return 0

        lax.fori_loop(0, _B // _G, grp_body, 0)
        qst_ref[:, :_C] = h_ref[...]
        qst_ref[:, _C:] = r_ref[...]

    out_ref[...] = qst_ref[...]


@jax.jit
def kernel(x, batch, W_ih, W_hh, b_ih, b_hh):
    n_pad = ((_N + _RB - 1) // _RB) * _RB
    x_pad = jnp.pad(x, ((0, n_pad - _N), (0, 0)))
    offs = jnp.searchsorted(batch, jnp.arange(_B + 1, dtype=jnp.int32),
                            side="left").astype(jnp.int32)
    lo_g = offs[:_B].reshape(_B // _G, _G)
    hi_g = offs[1:_B + 1].reshape(_B // _G, _G)
    wih_t = W_ih.T
    whh_t = W_hh.T

    return pl.pallas_call(
        _set2set_body,
        out_shape=jax.ShapeDtypeStruct((_B, 2 * _C), jnp.float32),
        in_specs=[
            pl.BlockSpec(memory_space=pltpu.SMEM),
            pl.BlockSpec(memory_space=pltpu.VMEM),
            pl.BlockSpec(memory_space=pltpu.VMEM),
            pl.BlockSpec(memory_space=pltpu.VMEM),
            pl.BlockSpec(memory_space=pltpu.VMEM),
            pl.BlockSpec(memory_space=pltpu.VMEM),
            pl.BlockSpec(memory_space=pltpu.VMEM),
            pl.BlockSpec(memory_space=pltpu.VMEM),
        ],
        out_specs=pl.BlockSpec(memory_space=pltpu.VMEM),
        scratch_shapes=[
            pltpu.VMEM((_B, _C), jnp.float32),
            pltpu.VMEM((_B, _C), jnp.float32),
            pltpu.VMEM((_B, 2 * _C), jnp.float32),
            pltpu.VMEM((_B, _C), jnp.float32),
        ],
        compiler_params=pltpu.CompilerParams(
            vmem_limit_bytes=120 * 1024 * 1024,
        ),
    )(offs, x_pad, lo_g, hi_g, wih_t, whh_t, b_ih[None, :], b_hh[None, :])
